# baseline (speedup 1.0000x reference)
import functools

import jax
import jax.numpy as jnp
from jax import lax
from jax.experimental import pallas as pl
from jax.experimental.pallas import tpu as pltpu
from jax.experimental.pallas import tpu_sc as plsc

_NC, _NS = 2, 16
_NW = _NC * _NS
_B, _L, _V, _D = 4096, 200, 16, 16
_K = 4
_BW = _B // _NW
_LC = 20
_NCHUNK = _L // _LC
_NBG = _BW // 16

_sc_mesh = plsc.VectorSubcoreMesh(core_axis_name="c", subcore_axis_name="s")


@functools.partial(
    pl.kernel,
    out_type=jax.ShapeDtypeStruct((_L * _D, _B), jnp.float32),
    mesh=_sc_mesh,
    scratch_types=[
        pltpu.VMEM((_V * _K,), jnp.float32),
        pltpu.VMEM((_D * _K,), jnp.float32),
        pltpu.VMEM((_D,), jnp.float32),
        pltpu.VMEM((_V * _D,), jnp.float32),
        pltpu.VMEM((_L, _BW), jnp.int32),
        pltpu.VMEM((_LC * _D, _BW), jnp.float32),
        pltpu.VMEM((_LC * _D, _BW), jnp.float32),
        pltpu.SemaphoreType.DMA,
        pltpu.SemaphoreType.DMA,
    ],
    compiler_params=pltpu.CompilerParams(needs_layout_passes=False),
)
def _sc_gather(table_hbm, w_hbm, b_hbm, ids_hbm, out_hbm, table_v, w_v, b_v,
               fused_v, ids_v, buf0_v, buf1_v, sem0, sem1):
    wid = lax.axis_index("s") * _NC + lax.axis_index("c")
    col0 = wid * _BW
    pltpu.sync_copy(table_hbm, table_v)
    pltpu.sync_copy(w_hbm, w_v)
    pltpu.sync_copy(b_hbm, b_v)
    pltpu.sync_copy(ids_hbm.at[:, pl.ds(col0, _BW)], ids_v)

    iota = lax.iota(jnp.int32, 16)
    tcols = [plsc.load_gather(table_v, [iota * _K + k]) for k in range(_K)]
    for d in range(_D):
        acc = plsc.load_gather(b_v, [jnp.full((16,), d, jnp.int32)])
        for k in range(_K):
            wdk = plsc.load_gather(
                w_v, [jnp.full((16,), d * _K + k, jnp.int32)]
            )
            acc = acc + tcols[k] * wdk
        fused_v[pl.ds(d * _V, _V)] = acc

    bufs = (buf0_v, buf1_v)
    sems = (sem0, sem1)

    @pl.loop(0, _NCHUNK // 2)
    def _pair(di):
        for half in range(2):
            ci = di * 2 + half
            buf_v = bufs[half]

            @pl.when(di > 0)
            def _drain(half=half, buf_v=buf_v):
                pltpu.make_async_copy(
                    out_hbm.at[pl.ds(0, _LC * _D), pl.ds(col0, _BW)],
                    buf_v,
                    sems[half],
                ).wait()

            @plsc.parallel_loop(0, _LC, unroll=2)
            def _pos(i, ci=ci, buf_v=buf_v):
                l = ci * _LC + i
                for bg in range(_NBG):
                    idsv = ids_v[l, pl.ds(bg * 16, 16)]
                    for d in range(_D):
                        col = plsc.load_gather(fused_v, [idsv + d * 16])
                        buf_v[i * _D + d, pl.ds(bg * 16, 16)] = col

            pltpu.async_copy(
                buf_v,
                out_hbm.at[
                    pl.ds(ci * (_LC * _D), _LC * _D), pl.ds(col0, _BW)
                ],
                sems[half],
            )

    for half in range(2):
        pltpu.make_async_copy(
            out_hbm.at[pl.ds(0, _LC * _D), pl.ds(col0, _BW)],
            bufs[half],
            sems[half],
        ).wait()


def kernel(input_ids, table, W, b):
    ids_t = input_ids.T.astype(jnp.int32)
    out = _sc_gather(
        table.reshape(_V * _K), W.reshape(_D * _K), b, ids_t
    )
    return out.reshape(_L, _D, _B).transpose(2, 0, 1)


# speedup vs baseline: 1.0953x; 1.0953x over previous
import functools

import jax
import jax.numpy as jnp
from jax import lax
from jax.experimental import pallas as pl
from jax.experimental.pallas import tpu as pltpu
from jax.experimental.pallas import tpu_sc as plsc

_NC, _NS = 2, 16
_NW = _NC * _NS
_B, _L, _V, _D = 4096, 200, 16, 16
_K = 4
_BW = _B // _NW
_LC = 20
_NCHUNK = _L // _LC
_NBG = _BW // 16

_sc_mesh = plsc.VectorSubcoreMesh(core_axis_name="c", subcore_axis_name="s")


@functools.partial(
    pl.kernel,
    out_type=jax.ShapeDtypeStruct((_L * _D, _B), jnp.float32),
    mesh=_sc_mesh,
    scratch_types=[
        pltpu.VMEM((_V * _K,), jnp.float32),
        pltpu.VMEM((16 + _D * _K,), jnp.float32),
        pltpu.VMEM((16 + _D,), jnp.float32),
        pltpu.VMEM((_V * _D,), jnp.float32),
        pltpu.VMEM((_L, _BW), jnp.int32),
        pltpu.VMEM((_LC * _D, _BW), jnp.float32),
        pltpu.VMEM((_LC * _D, _BW), jnp.float32),
        pltpu.SemaphoreType.DMA,
        pltpu.SemaphoreType.DMA,
    ],
    compiler_params=pltpu.CompilerParams(needs_layout_passes=False),
)
def _sc_gather(table_hbm, w_hbm, b_hbm, ids_hbm, out_hbm, table_v, w_v, b_v,
               fused_v, ids_v, buf0_v, buf1_v, sem0, sem1):
    wid = lax.axis_index("s") * _NC + lax.axis_index("c")
    col0 = wid * _BW
    pltpu.sync_copy(table_hbm, table_v)
    pltpu.sync_copy(w_hbm, w_v.at[pl.ds(16, _D * _K)])
    pltpu.sync_copy(b_hbm, b_v.at[pl.ds(16, _D)])
    pltpu.sync_copy(ids_hbm.at[:, pl.ds(col0, _BW)], ids_v)

    iota = lax.iota(jnp.int32, 16)
    tcols = [plsc.load_gather(table_v, [iota * _K + k]) for k in range(_K)]
    for d in range(_D):
        acc = plsc.load_gather(b_v, [jnp.full((16,), 16 + d, jnp.int32)])
        for k in range(_K):
            wdk = plsc.load_gather(
                w_v, [jnp.full((16,), 16 + d * _K + k, jnp.int32)]
            )
            acc = acc + tcols[k] * wdk
        fused_v[pl.ds(d * _V, _V)] = acc

    bufs = (buf0_v, buf1_v)
    sems = (sem0, sem1)

    @pl.loop(0, _NCHUNK // 2)
    def _pair(di):
        for half in range(2):
            ci = di * 2 + half
            buf_v = bufs[half]

            @pl.when(di > 0)
            def _drain(half=half, buf_v=buf_v):
                pltpu.make_async_copy(
                    out_hbm.at[pl.ds(0, _LC * _D), pl.ds(col0, _BW)],
                    buf_v,
                    sems[half],
                ).wait()

            @plsc.parallel_loop(0, _LC, unroll=1)
            def _pos(i, ci=ci, buf_v=buf_v):
                l = ci * _LC + i
                for bg in range(_NBG):
                    idsv = ids_v[l, pl.ds(bg * 16, 16)]
                    for d in range(_D):
                        col = plsc.load_gather(fused_v, [idsv + d * 16])
                        buf_v[i * _D + d, pl.ds(bg * 16, 16)] = col

            pltpu.async_copy(
                buf_v,
                out_hbm.at[
                    pl.ds(ci * (_LC * _D), _LC * _D), pl.ds(col0, _BW)
                ],
                sems[half],
            )

    for half in range(2):
        pltpu.make_async_copy(
            out_hbm.at[pl.ds(0, _LC * _D), pl.ds(col0, _BW)],
            bufs[half],
            sems[half],
        ).wait()


def kernel(input_ids, table, W, b):
    ids_t = input_ids.T.astype(jnp.int32)
    out = _sc_gather(
        table.reshape(_V * _K), W.reshape(_D * _K), b, ids_t
    )
    return out.reshape(_L, _D, _B).transpose(2, 0, 1)
